# Initial kernel scaffold; baseline (speedup 1.0000x reference)
import jax
import jax.numpy as jnp
from jax.experimental import pallas as pl


def kernel(x, edge_index, batch, params):
    raise NotImplementedError("write your pallas kernel here")


# speedup vs baseline: 1.4643x; 1.4643x over previous
import functools

import jax
import jax.numpy as jnp
from jax import lax
from jax.experimental import pallas as pl
from jax.experimental.pallas import tpu as pltpu
from jax.experimental.pallas import tpu_sc as plsc

_N = 10000
_E = 320000
_D = 128
_G = 64

_RB = 1000
_NB = _N // _RB

_NC = 2
_NS = 16
_NW = _NC * _NS
_EPT = _E // _NW
_EC = 80
_NCHUNK = _EPT // _EC
_ZR = 80
_NZ = _N // _ZR
_ZT = -(-_NZ // _NS)


def _edge_sc_body(k_hbm, qv_hbm, src_hbm, dst_hbm, z_hbm, out_hbm,
                  sidx, didx, kbuf, qvbuf, mbuf, acc, sem_k, sem_qv):
    cid = lax.axis_index("c")
    sid = lax.axis_index("s")
    wid = sid * _NC + cid

    pltpu.sync_copy(z_hbm, mbuf)
    for t in range(_ZT):
        ci = sid + _NS * t

        @pl.when(ci < _NZ)
        def _():
            off = pl.multiple_of(ci * _ZR, 8)
            pltpu.sync_copy(mbuf, acc.at[pl.ds(off, _ZR)])

    plsc.subcore_barrier()

    ebase = wid * _EPT

    def chunk(c, carry):
        base = pl.multiple_of(ebase + c * _EC, 8)
        pltpu.sync_copy(src_hbm.at[pl.ds(base, _EC)], sidx)
        pltpu.sync_copy(dst_hbm.at[pl.ds(base, _EC)], didx)
        cp_k = pltpu.async_copy(k_hbm.at[didx], kbuf, sem_k)
        cp_qv = pltpu.async_copy(qv_hbm.at[sidx], qvbuf, sem_qv)
        cp_k.wait()
        cp_qv.wait()

        def row(r, rc):
            for j in range(_D // 16):
                kk = kbuf[r, pl.ds(16 * j, 16)]
                qq = qvbuf[r, pl.ds(16 * j, 16)]
                vv = qvbuf[r, pl.ds(_D + 16 * j, 16)]
                eta = 1.0 / (1.0 + jnp.exp(-(kk + qq)))
                mbuf[r, pl.ds(16 * j, 16)] = eta * vv
            return rc

        lax.fori_loop(0, _EC, row, 0)
        pltpu.sync_copy(mbuf, acc.at[didx], add=True)
        return carry

    lax.fori_loop(0, _NCHUNK, chunk, 0)
    plsc.subcore_barrier()

    for t in range(_ZT):
        ci = sid + _NS * t

        @pl.when(ci < _NZ)
        def _():
            off = pl.multiple_of(ci * _ZR, 8)
            pltpu.sync_copy(acc.at[pl.ds(off, _ZR)], mbuf)
            pltpu.sync_copy(mbuf, out_hbm.at[cid, pl.ds(off, _ZR)])


@functools.cache
def _edge_sc_kernel():
    return pl.kernel(
        _edge_sc_body,
        out_type=jax.ShapeDtypeStruct((_NC, _N, _D), jnp.float32),
        mesh=plsc.VectorSubcoreMesh(core_axis_name="c", subcore_axis_name="s",
                                    num_cores=_NC, num_subcores=_NS),
        scratch_types=[
            pltpu.VMEM((_EC,), jnp.int32),
            pltpu.VMEM((_EC,), jnp.int32),
            pltpu.VMEM((_EC, _D), jnp.float32),
            pltpu.VMEM((_EC, 2 * _D), jnp.float32),
            pltpu.VMEM((_EC, _D), jnp.float32),
            pltpu.VMEM_SHARED((_N, _D), jnp.float32),
            pltpu.SemaphoreType.DMA,
            pltpu.SemaphoreType.DMA,
        ],
    )


def _edge_sc(k, qv, src, dst, zrows):
    return _edge_sc_kernel()(k, qv, src, dst, zrows)


def _proj_body(fused, h_ref, wk_ref, wqv_ref, ws_ref, bk_ref, bqv_ref, bs_ref,
               *rest):
    if fused:
        stats_ref, g_ref, bb_ref = rest[0], rest[1], rest[2]
        k_ref, qv_ref, s_ref = rest[3], rest[4], rest[5]
    else:
        k_ref, qv_ref, s_ref = rest[0], rest[1], rest[2]
    h = h_ref[...]
    if fused:
        mean = stats_ref[0:1, :] * (1.0 / _N)
        var = stats_ref[1:2, :] * (1.0 / _N) - mean * mean
        sc = g_ref[...] * lax.rsqrt(var + 1e-5)
        h = (h - mean) * sc + bb_ref[...]
    k_ref[...] = jnp.dot(h, wk_ref[...], preferred_element_type=jnp.float32) + bk_ref[...]
    qv_ref[...] = jnp.dot(h, wqv_ref[...], preferred_element_type=jnp.float32) + bqv_ref[...]
    s_ref[...] = jnp.dot(h, ws_ref[...], preferred_element_type=jnp.float32) + bs_ref[...]


def _proj(h, wk, wqv, ws, bk, bqv, bs, stats=None, g=None, b=None):
    fused = stats is not None
    const = lambda i: (0, 0)
    in_specs = [
        pl.BlockSpec((_RB, _D), lambda i: (i, 0)),
        pl.BlockSpec((_D, _D), const),
        pl.BlockSpec((_D, 2 * _D), const),
        pl.BlockSpec((_D, _D), const),
        pl.BlockSpec((1, _D), const),
        pl.BlockSpec((1, 2 * _D), const),
        pl.BlockSpec((1, _D), const),
    ]
    ins = [h, wk, wqv, ws, bk, bqv, bs]
    if fused:
        in_specs += [pl.BlockSpec((2, _D), const),
                     pl.BlockSpec((1, _D), const),
                     pl.BlockSpec((1, _D), const)]
        ins += [stats, g, b]
    return pl.pallas_call(
        functools.partial(_proj_body, fused),
        grid=(_NB,),
        in_specs=in_specs,
        out_specs=[
            pl.BlockSpec((_RB, _D), lambda i: (i, 0)),
            pl.BlockSpec((_RB, 2 * _D), lambda i: (i, 0)),
            pl.BlockSpec((_RB, _D), lambda i: (i, 0)),
        ],
        out_shape=[
            jax.ShapeDtypeStruct((_N, _D), jnp.float32),
            jax.ShapeDtypeStruct((_N, 2 * _D), jnp.float32),
            jax.ShapeDtypeStruct((_N, _D), jnp.float32),
        ],
    )(*ins)


def _post_body(agg_ref, s_ref, t_ref, stats_ref):
    i = pl.program_id(0)
    t = jnp.maximum(agg_ref[0] + agg_ref[1] + s_ref[...], 0.0)
    t_ref[...] = t

    @pl.when(i == 0)
    def _():
        stats_ref[...] = jnp.zeros_like(stats_ref)

    stats_ref[...] += jnp.concatenate(
        [jnp.sum(t, axis=0, keepdims=True),
         jnp.sum(t * t, axis=0, keepdims=True)], axis=0)


def _post(agg, s):
    return pl.pallas_call(
        _post_body,
        grid=(_NB,),
        in_specs=[
            pl.BlockSpec((_NC, _RB, _D), lambda i: (0, i, 0)),
            pl.BlockSpec((_RB, _D), lambda i: (i, 0)),
        ],
        out_specs=[
            pl.BlockSpec((_RB, _D), lambda i: (i, 0)),
            pl.BlockSpec((2, _D), lambda i: (0, 0)),
        ],
        out_shape=[
            jax.ShapeDtypeStruct((_N, _D), jnp.float32),
            jax.ShapeDtypeStruct((2, _D), jnp.float32),
        ],
    )(agg, s)


def _bn_rows(x, g, b):
    m = jnp.mean(x, axis=0, keepdims=True)
    v = jnp.mean((x - m) * (x - m), axis=0, keepdims=True)
    return (x - m) * lax.rsqrt(v + 1e-5) * g + b


def _final_body(t_ref, stats_ref, bn_g_ref, bn_b_ref, batch_ref,
                gapg_ref, gapb_ref, gspg_ref, gspb_ref,
                w1_ref, b1_ref, lng_ref, lnb_ref, w2_ref, b2_ref,
                out_ref, gap_s, cnt_s, gsp_s):
    i = pl.program_id(0)
    mean = stats_ref[0:1, :] * (1.0 / _N)
    var = stats_ref[1:2, :] * (1.0 / _N) - mean * mean
    sc = bn_g_ref[...] * lax.rsqrt(var + 1e-5)
    h = (t_ref[...] - mean) * sc + bn_b_ref[...]
    bid = batch_ref[...]
    onehot = (bid == lax.broadcasted_iota(jnp.int32, (_RB, _G), 1)
              ).astype(jnp.float32)

    @pl.when(i == 0)
    def _():
        gap_s[...] = jnp.zeros_like(gap_s)
        cnt_s[...] = jnp.zeros_like(cnt_s)
        gsp_s[...] = jnp.full_like(gsp_s, jnp.finfo(jnp.float32).min)

    gap_s[...] += lax.dot_general(onehot, h, (((0,), (0,)), ((), ())),
                                  preferred_element_type=jnp.float32)
    cnt_s[...] += jnp.sum(onehot, axis=0, keepdims=True)

    def gloop(g, c):
        m = jnp.where(bid == g, h, jnp.finfo(jnp.float32).min)
        mx = jnp.max(m, axis=0, keepdims=True)
        gsp_s[pl.ds(g, 1), :] = jnp.maximum(gsp_s[pl.ds(g, 1), :], mx)
        return c

    lax.fori_loop(0, _G, gloop, 0)

    @pl.when(i == _NB - 1)
    def _():
        cnt = jnp.maximum(cnt_s[...].reshape(_G, 1), 1.0)
        gap = _bn_rows(gap_s[...] / cnt, gapg_ref[...], gapb_ref[...])
        gsp = _bn_rows(gsp_s[...], gspg_ref[...], gspb_ref[...])
        cat = jnp.concatenate([gap, gsp], axis=1)
        o = jnp.maximum(
            jnp.dot(cat, w1_ref[...], preferred_element_type=jnp.float32)
            + b1_ref[...], 0.0)
        o = _bn_rows(o, lng_ref[...], lnb_ref[...])
        out_ref[...] = (jnp.dot(o, w2_ref[...],
                                preferred_element_type=jnp.float32)
                        + b2_ref[...])


def _final(t, stats, bn_g, bn_b, batch2, gapg, gapb, gspg, gspb,
           w1, b1, lng, lnb, w2, b2):
    const = lambda i: (0, 0)
    return pl.pallas_call(
        _final_body,
        grid=(_NB,),
        in_specs=[
            pl.BlockSpec((_RB, _D), lambda i: (i, 0)),
            pl.BlockSpec((2, _D), const),
            pl.BlockSpec((1, _D), const),
            pl.BlockSpec((1, _D), const),
            pl.BlockSpec((_RB, 1), lambda i: (i, 0)),
            pl.BlockSpec((1, _D), const),
            pl.BlockSpec((1, _D), const),
            pl.BlockSpec((1, _D), const),
            pl.BlockSpec((1, _D), const),
            pl.BlockSpec((2 * _D, _D), const),
            pl.BlockSpec((1, _D), const),
            pl.BlockSpec((1, _D), const),
            pl.BlockSpec((1, _D), const),
            pl.BlockSpec((_D, _D), const),
            pl.BlockSpec((1, _D), const),
        ],
        out_specs=pl.BlockSpec((_G, _D), const),
        out_shape=jax.ShapeDtypeStruct((_G, _D), jnp.float32),
        scratch_shapes=[
            pltpu.VMEM((_G, _D), jnp.float32),
            pltpu.VMEM((1, _G), jnp.float32),
            pltpu.VMEM((_G, _D), jnp.float32),
        ],
    )(t, stats, bn_g, bn_b, batch2, gapg, gapb, gspg, gspb,
      w1, b1, lng, lnb, w2, b2)


def _layer_params(p, l):
    wk = p[f'conv{l}_Wk']
    wqv = jnp.concatenate([p[f'conv{l}_Wq'], p[f'conv{l}_Wv']], axis=1)
    ws = p[f'conv{l}_Ws']
    bk = p[f'conv{l}_bk'].reshape(1, _D)
    bqv = jnp.concatenate([p[f'conv{l}_bq'], p[f'conv{l}_bv']]).reshape(1, 2 * _D)
    bs = p[f'conv{l}_b'].reshape(1, _D)
    return wk, wqv, ws, bk, bqv, bs


def kernel(x, edge_index, batch, params):
    p = params
    src = edge_index[0]
    dst = edge_index[1]
    batch2 = batch.reshape(_N, 1)
    zrows = jnp.zeros((_ZR, _D), jnp.float32)

    k0, qv0, s0 = _proj(x, *_layer_params(p, 0))
    agg0 = _edge_sc(k0, qv0, src, dst, zrows)
    t0, st0 = _post(agg0, s0)

    k1, qv1, s1 = _proj(t0, *_layer_params(p, 1), stats=st0,
                        g=p['bn0_g'].reshape(1, _D), b=p['bn0_b'].reshape(1, _D))
    agg1 = _edge_sc(k1, qv1, src, dst, zrows)
    t1, st1 = _post(agg1, s1)

    w2 = jnp.zeros((_D, _D), jnp.float32).at[:, :10].set(p['last_W'])
    b2 = jnp.zeros((1, _D), jnp.float32).at[0, :10].set(p['last_b'])
    out = _final(t1, st1,
                 p['bn1_g'].reshape(1, _D), p['bn1_b'].reshape(1, _D),
                 batch2,
                 p['gap_g'].reshape(1, _D), p['gap_b'].reshape(1, _D),
                 p['gsp_g'].reshape(1, _D), p['gsp_b'].reshape(1, _D),
                 p['hl0_W'], p['hl0_b'].reshape(1, _D),
                 p['hln0_g'].reshape(1, _D), p['hln0_b'].reshape(1, _D),
                 w2, b2)
    return out[:, :10]


# speedup vs baseline: 1.7252x; 1.1782x over previous
import functools

import jax
import jax.numpy as jnp
from jax import lax
from jax.experimental import pallas as pl
from jax.experimental.pallas import tpu as pltpu
from jax.experimental.pallas import tpu_sc as plsc

_N = 10000
_E = 320000
_D = 128
_G = 64

_RB = 1000
_NB = _N // _RB

_NC = 2
_NS = 16
_NW = _NC * _NS
_EPT = _E // _NW
_EC = 40
_NCHUNK = _EPT // _EC
_GC = 50
_NG = _NCHUNK // _GC
_ZR = 40
_NZ = _N // _ZR
_ZT = -(-_NZ // _NS)


def _edge_sc_body(k_hbm, qv_hbm, src_hbm, dst_hbm, z_hbm, out_hbm,
                  sidx, didx, kb0, kb1, qb0, qb1, acc, sg0, sg1, ss0, ss1):
    cid = lax.axis_index("c")
    sid = lax.axis_index("s")
    wid = sid * _NC + cid
    kb = (kb0, kb1)
    qb = (qb0, qb1)
    sg = (sg0, sg1)
    ss = (ss0, ss1)

    pltpu.sync_copy(z_hbm, kb0)
    for t in range(_ZT):
        ci = sid + _NS * t

        @pl.when(ci < _NZ)
        def _():
            off = pl.multiple_of(ci * _ZR, 8)
            pltpu.sync_copy(kb0, acc.at[pl.ds(off, _ZR)])

    plsc.subcore_barrier()

    def issue_gather(c, p):
        pltpu.async_copy(k_hbm.at[didx.at[c]], kb[p], sg[p])
        pltpu.async_copy(qv_hbm.at[sidx.at[c]], qb[p], sg[p])

    def drain_gather(c, p):
        pltpu.make_async_copy(k_hbm.at[didx.at[c]], kb[p], sg[p]).wait()
        pltpu.make_async_copy(qv_hbm.at[sidx.at[c]], qb[p], sg[p]).wait()

    def drain_scatter(c, p):
        pltpu.make_async_copy(kb[p], acc.at[didx.at[c]], ss[p]).wait()

    def compute(p):
        kbp, qbp = kb[p], qb[p]

        def row(r, rc):
            for j in range(_D // 16):
                kk = kbp[r, pl.ds(16 * j, 16)]
                qq = qbp[r, pl.ds(16 * j, 16)]
                vv = qbp[r, pl.ds(_D + 16 * j, 16)]
                kbp[r, pl.ds(16 * j, 16)] = vv / (1.0 + jnp.exp(-(kk + qq)))
            return rc

        lax.fori_loop(0, _EC, row, 0)

    def pair(t, carry):
        for p in range(2):
            c = 2 * t + p

            @pl.when(c > 0)
            def _():
                drain_scatter(c - 1, 1 - p)

            @pl.when(c + 1 < _GC)
            def _():
                issue_gather(c + 1, 1 - p)

            drain_gather(c, p)
            compute(p)
            pltpu.async_copy(kb[p], acc.at[didx.at[c]], ss[p], add=True)
        return carry

    for g in range(_NG):
        pltpu.sync_copy(src_hbm.at[wid, g], sidx)
        pltpu.sync_copy(dst_hbm.at[wid, g], didx)
        issue_gather(0, 0)
        lax.fori_loop(0, _GC // 2, pair, 0)
        drain_scatter(_GC - 1, 1)

    plsc.subcore_barrier()

    for t in range(_ZT):
        ci = sid + _NS * t

        @pl.when(ci < _NZ)
        def _():
            off = pl.multiple_of(ci * _ZR, 8)
            pltpu.sync_copy(acc.at[pl.ds(off, _ZR)], kb0)
            pltpu.sync_copy(kb0, out_hbm.at[cid, pl.ds(off, _ZR)])


@functools.cache
def _edge_sc_kernel():
    return pl.kernel(
        _edge_sc_body,
        out_type=jax.ShapeDtypeStruct((_NC, _N, _D), jnp.float32),
        mesh=plsc.VectorSubcoreMesh(core_axis_name="c", subcore_axis_name="s",
                                    num_cores=_NC, num_subcores=_NS),
        scratch_types=[
            pltpu.VMEM((_GC, _EC), jnp.int32),
            pltpu.VMEM((_GC, _EC), jnp.int32),
            pltpu.VMEM((_EC, _D), jnp.float32),
            pltpu.VMEM((_EC, _D), jnp.float32),
            pltpu.VMEM((_EC, 2 * _D), jnp.float32),
            pltpu.VMEM((_EC, 2 * _D), jnp.float32),
            pltpu.VMEM_SHARED((_N, _D), jnp.float32),
            pltpu.SemaphoreType.DMA,
            pltpu.SemaphoreType.DMA,
            pltpu.SemaphoreType.DMA,
            pltpu.SemaphoreType.DMA,
        ],
    )


def _edge_sc(k, qv, src, dst, zrows):
    src4 = src.reshape(_NW, _NG, _GC, _EC)
    dst4 = dst.reshape(_NW, _NG, _GC, _EC)
    return _edge_sc_kernel()(k, qv, src4, dst4, zrows)


def _proj_body(fused, h_ref, wk_ref, wqv_ref, ws_ref, bk_ref, bqv_ref, bs_ref,
               *rest):
    if fused:
        stats_ref, g_ref, bb_ref = rest[0], rest[1], rest[2]
        k_ref, qv_ref, s_ref = rest[3], rest[4], rest[5]
    else:
        k_ref, qv_ref, s_ref = rest[0], rest[1], rest[2]
    h = h_ref[...]
    if fused:
        mean = stats_ref[0:1, :] * (1.0 / _N)
        var = stats_ref[1:2, :] * (1.0 / _N) - mean * mean
        sc = g_ref[...] * lax.rsqrt(var + 1e-5)
        h = (h - mean) * sc + bb_ref[...]
    k_ref[...] = jnp.dot(h, wk_ref[...], preferred_element_type=jnp.float32) + bk_ref[...]
    qv_ref[...] = jnp.dot(h, wqv_ref[...], preferred_element_type=jnp.float32) + bqv_ref[...]
    s_ref[...] = jnp.dot(h, ws_ref[...], preferred_element_type=jnp.float32) + bs_ref[...]


def _proj(h, wk, wqv, ws, bk, bqv, bs, stats=None, g=None, b=None):
    fused = stats is not None
    const = lambda i: (0, 0)
    in_specs = [
        pl.BlockSpec((_RB, _D), lambda i: (i, 0)),
        pl.BlockSpec((_D, _D), const),
        pl.BlockSpec((_D, 2 * _D), const),
        pl.BlockSpec((_D, _D), const),
        pl.BlockSpec((1, _D), const),
        pl.BlockSpec((1, 2 * _D), const),
        pl.BlockSpec((1, _D), const),
    ]
    ins = [h, wk, wqv, ws, bk, bqv, bs]
    if fused:
        in_specs += [pl.BlockSpec((2, _D), const),
                     pl.BlockSpec((1, _D), const),
                     pl.BlockSpec((1, _D), const)]
        ins += [stats, g, b]
    return pl.pallas_call(
        functools.partial(_proj_body, fused),
        grid=(_NB,),
        in_specs=in_specs,
        out_specs=[
            pl.BlockSpec((_RB, _D), lambda i: (i, 0)),
            pl.BlockSpec((_RB, 2 * _D), lambda i: (i, 0)),
            pl.BlockSpec((_RB, _D), lambda i: (i, 0)),
        ],
        out_shape=[
            jax.ShapeDtypeStruct((_N, _D), jnp.float32),
            jax.ShapeDtypeStruct((_N, 2 * _D), jnp.float32),
            jax.ShapeDtypeStruct((_N, _D), jnp.float32),
        ],
    )(*ins)


def _post_body(agg_ref, s_ref, t_ref, stats_ref):
    i = pl.program_id(0)
    t = jnp.maximum(agg_ref[0] + agg_ref[1] + s_ref[...], 0.0)
    t_ref[...] = t

    @pl.when(i == 0)
    def _():
        stats_ref[...] = jnp.zeros_like(stats_ref)

    stats_ref[...] += jnp.concatenate(
        [jnp.sum(t, axis=0, keepdims=True),
         jnp.sum(t * t, axis=0, keepdims=True)], axis=0)


def _post(agg, s):
    return pl.pallas_call(
        _post_body,
        grid=(_NB,),
        in_specs=[
            pl.BlockSpec((_NC, _RB, _D), lambda i: (0, i, 0)),
            pl.BlockSpec((_RB, _D), lambda i: (i, 0)),
        ],
        out_specs=[
            pl.BlockSpec((_RB, _D), lambda i: (i, 0)),
            pl.BlockSpec((2, _D), lambda i: (0, 0)),
        ],
        out_shape=[
            jax.ShapeDtypeStruct((_N, _D), jnp.float32),
            jax.ShapeDtypeStruct((2, _D), jnp.float32),
        ],
    )(agg, s)


def _bn_rows(x, g, b):
    m = jnp.mean(x, axis=0, keepdims=True)
    v = jnp.mean((x - m) * (x - m), axis=0, keepdims=True)
    return (x - m) * lax.rsqrt(v + 1e-5) * g + b


def _final_body(t_ref, stats_ref, bn_g_ref, bn_b_ref, batch_ref,
                gapg_ref, gapb_ref, gspg_ref, gspb_ref,
                w1_ref, b1_ref, lng_ref, lnb_ref, w2_ref, b2_ref,
                out_ref, gap_s, cnt_s, gsp_s):
    i = pl.program_id(0)
    mean = stats_ref[0:1, :] * (1.0 / _N)
    var = stats_ref[1:2, :] * (1.0 / _N) - mean * mean
    sc = bn_g_ref[...] * lax.rsqrt(var + 1e-5)
    h = (t_ref[...] - mean) * sc + bn_b_ref[...]
    bid = batch_ref[...]
    onehot = (bid == lax.broadcasted_iota(jnp.int32, (_RB, _G), 1)
              ).astype(jnp.float32)

    @pl.when(i == 0)
    def _():
        gap_s[...] = jnp.zeros_like(gap_s)
        cnt_s[...] = jnp.zeros_like(cnt_s)
        gsp_s[...] = jnp.full_like(gsp_s, jnp.finfo(jnp.float32).min)

    gap_s[...] += lax.dot_general(onehot, h, (((0,), (0,)), ((), ())),
                                  preferred_element_type=jnp.float32)
    cnt_s[...] += jnp.sum(onehot, axis=0, keepdims=True)

    def gloop(g, c):
        m = jnp.where(bid == g, h, jnp.finfo(jnp.float32).min)
        mx = jnp.max(m, axis=0, keepdims=True)
        gsp_s[pl.ds(g, 1), :] = jnp.maximum(gsp_s[pl.ds(g, 1), :], mx)
        return c

    lax.fori_loop(0, _G, gloop, 0)

    @pl.when(i == _NB - 1)
    def _():
        cnt = jnp.maximum(cnt_s[...].reshape(_G, 1), 1.0)
        gap = _bn_rows(gap_s[...] / cnt, gapg_ref[...], gapb_ref[...])
        gsp = _bn_rows(gsp_s[...], gspg_ref[...], gspb_ref[...])
        cat = jnp.concatenate([gap, gsp], axis=1)
        o = jnp.maximum(
            jnp.dot(cat, w1_ref[...], preferred_element_type=jnp.float32)
            + b1_ref[...], 0.0)
        o = _bn_rows(o, lng_ref[...], lnb_ref[...])
        out_ref[...] = (jnp.dot(o, w2_ref[...],
                                preferred_element_type=jnp.float32)
                        + b2_ref[...])


def _final(t, stats, bn_g, bn_b, batch2, gapg, gapb, gspg, gspb,
           w1, b1, lng, lnb, w2, b2):
    const = lambda i: (0, 0)
    return pl.pallas_call(
        _final_body,
        grid=(_NB,),
        in_specs=[
            pl.BlockSpec((_RB, _D), lambda i: (i, 0)),
            pl.BlockSpec((2, _D), const),
            pl.BlockSpec((1, _D), const),
            pl.BlockSpec((1, _D), const),
            pl.BlockSpec((_RB, 1), lambda i: (i, 0)),
            pl.BlockSpec((1, _D), const),
            pl.BlockSpec((1, _D), const),
            pl.BlockSpec((1, _D), const),
            pl.BlockSpec((1, _D), const),
            pl.BlockSpec((2 * _D, _D), const),
            pl.BlockSpec((1, _D), const),
            pl.BlockSpec((1, _D), const),
            pl.BlockSpec((1, _D), const),
            pl.BlockSpec((_D, _D), const),
            pl.BlockSpec((1, _D), const),
        ],
        out_specs=pl.BlockSpec((_G, _D), const),
        out_shape=jax.ShapeDtypeStruct((_G, _D), jnp.float32),
        scratch_shapes=[
            pltpu.VMEM((_G, _D), jnp.float32),
            pltpu.VMEM((1, _G), jnp.float32),
            pltpu.VMEM((_G, _D), jnp.float32),
        ],
    )(t, stats, bn_g, bn_b, batch2, gapg, gapb, gspg, gspb,
      w1, b1, lng, lnb, w2, b2)


def _layer_params(p, l):
    wk = p[f'conv{l}_Wk']
    wqv = jnp.concatenate([p[f'conv{l}_Wq'], p[f'conv{l}_Wv']], axis=1)
    ws = p[f'conv{l}_Ws']
    bk = p[f'conv{l}_bk'].reshape(1, _D)
    bqv = jnp.concatenate([p[f'conv{l}_bq'], p[f'conv{l}_bv']]).reshape(1, 2 * _D)
    bs = p[f'conv{l}_b'].reshape(1, _D)
    return wk, wqv, ws, bk, bqv, bs


def kernel(x, edge_index, batch, params):
    p = params
    src = edge_index[0]
    dst = edge_index[1]
    batch2 = batch.reshape(_N, 1)
    zrows = jnp.zeros((_ZR, _D), jnp.float32)

    k0, qv0, s0 = _proj(x, *_layer_params(p, 0))
    agg0 = _edge_sc(k0, qv0, src, dst, zrows)
    t0, st0 = _post(agg0, s0)

    k1, qv1, s1 = _proj(t0, *_layer_params(p, 1), stats=st0,
                        g=p['bn0_g'].reshape(1, _D), b=p['bn0_b'].reshape(1, _D))
    agg1 = _edge_sc(k1, qv1, src, dst, zrows)
    t1, st1 = _post(agg1, s1)

    w2 = jnp.zeros((_D, _D), jnp.float32).at[:, :10].set(p['last_W'])
    b2 = jnp.zeros((1, _D), jnp.float32).at[0, :10].set(p['last_b'])
    out = _final(t1, st1,
                 p['bn1_g'].reshape(1, _D), p['bn1_b'].reshape(1, _D),
                 batch2,
                 p['gap_g'].reshape(1, _D), p['gap_b'].reshape(1, _D),
                 p['gsp_g'].reshape(1, _D), p['gsp_b'].reshape(1, _D),
                 p['hl0_W'], p['hl0_b'].reshape(1, _D),
                 p['hln0_g'].reshape(1, _D), p['hln0_b'].reshape(1, _D),
                 w2, b2)
    return out[:, :10]


# speedup vs baseline: 1.7775x; 1.0303x over previous
import functools

import jax
import jax.numpy as jnp
from jax import lax
from jax.experimental import pallas as pl
from jax.experimental.pallas import tpu as pltpu
from jax.experimental.pallas import tpu_sc as plsc

_N = 10000
_E = 320000
_D = 128
_G = 64

_RB = 1000
_NB = _N // _RB

_NC = 2
_NS = 16
_NW = _NC * _NS
_EPT = _E // _NW
_EC = 40
_NCHUNK = _EPT // _EC
_GC = 50
_NG = _NCHUNK // _GC
_ZR = 40
_NZ = _N // _ZR
_ZT = -(-_NZ // _NS)


def _edge_sc_body(k_hbm, qv_hbm, src_hbm, dst_hbm, z_hbm, out_hbm,
                  sidx, didx, kb0, kb1, kb2, qb0, qb1, acc,
                  sg0, sg1, ss0, ss1, ss2):
    cid = lax.axis_index("c")
    sid = lax.axis_index("s")
    wid = sid * _NC + cid
    kb = (kb0, kb1, kb2)
    qb = (qb0, qb1)
    sg = (sg0, sg1)
    ss = (ss0, ss1, ss2)

    pltpu.sync_copy(z_hbm, kb0)
    for t in range(_ZT):
        ci = sid + _NS * t

        @pl.when(ci < _NZ)
        def _():
            off = pl.multiple_of(ci * _ZR, 8)
            pltpu.sync_copy(kb0, acc.at[pl.ds(off, _ZR)])

    plsc.subcore_barrier()

    def issue_gather(c, m, p):
        pltpu.async_copy(k_hbm.at[didx.at[c]], kb[m], sg[p])
        pltpu.async_copy(qv_hbm.at[sidx.at[c]], qb[p], sg[p])

    def drain_gather(c, m, p):
        pltpu.make_async_copy(k_hbm.at[didx.at[c]], kb[m], sg[p]).wait()
        pltpu.make_async_copy(qv_hbm.at[sidx.at[c]], qb[p], sg[p]).wait()

    def drain_scatter(c, m):
        pltpu.make_async_copy(kb[m], acc.at[didx.at[c]], ss[m]).wait()

    def compute(m, p):
        kbp, qbp = kb[m], qb[p]

        def row(r, rc):
            for j in range(_D // 16):
                kk = kbp[r, pl.ds(16 * j, 16)]
                qq = qbp[r, pl.ds(16 * j, 16)]
                vv = qbp[r, pl.ds(_D + 16 * j, 16)]
                kbp[r, pl.ds(16 * j, 16)] = vv / (1.0 + jnp.exp(-(kk + qq)))
            return rc

        lax.fori_loop(0, _EC, row, 0)

    def step(c, m, p, issue_next, guard_drain):
        mm = (m + 1) % 3

        def _drain_prev():
            drain_scatter(c - 2, mm)

        if guard_drain:
            pl.when(c >= 2)(_drain_prev)
        else:
            _drain_prev()
        if issue_next:
            issue_gather(c + 1, mm, 1 - p)
        drain_gather(c, m, p)
        compute(m, p)
        pltpu.async_copy(kb[m], acc.at[didx.at[c]], ss[m], add=True)

    def six(t, carry):
        for u in range(6):
            step(6 * t + u, u % 3, u % 2, True, True)
        return carry

    for g in range(_NG):
        pltpu.sync_copy(src_hbm.at[wid, g], sidx)
        pltpu.sync_copy(dst_hbm.at[wid, g], didx)
        issue_gather(0, 0, 0)
        lax.fori_loop(0, (_GC - 2) // 6, six, 0)
        step(_GC - 2, (_GC - 2) % 3, (_GC - 2) % 2, True, False)
        step(_GC - 1, (_GC - 1) % 3, (_GC - 1) % 2, False, False)
        drain_scatter(_GC - 2, (_GC - 2) % 3)
        drain_scatter(_GC - 1, (_GC - 1) % 3)

    plsc.subcore_barrier()

    for t in range(_ZT):
        ci = sid + _NS * t

        @pl.when(ci < _NZ)
        def _():
            off = pl.multiple_of(ci * _ZR, 8)
            pltpu.sync_copy(acc.at[pl.ds(off, _ZR)], kb0)
            pltpu.sync_copy(kb0, out_hbm.at[cid, pl.ds(off, _ZR)])


@functools.cache
def _edge_sc_kernel():
    return pl.kernel(
        _edge_sc_body,
        out_type=jax.ShapeDtypeStruct((_NC, _N, _D), jnp.float32),
        mesh=plsc.VectorSubcoreMesh(core_axis_name="c", subcore_axis_name="s",
                                    num_cores=_NC, num_subcores=_NS),
        scratch_types=[
            pltpu.VMEM((_GC, _EC), jnp.int32),
            pltpu.VMEM((_GC, _EC), jnp.int32),
            pltpu.VMEM((_EC, _D), jnp.float32),
            pltpu.VMEM((_EC, _D), jnp.float32),
            pltpu.VMEM((_EC, _D), jnp.float32),
            pltpu.VMEM((_EC, 2 * _D), jnp.float32),
            pltpu.VMEM((_EC, 2 * _D), jnp.float32),
            pltpu.VMEM_SHARED((_N, _D), jnp.float32),
            pltpu.SemaphoreType.DMA,
            pltpu.SemaphoreType.DMA,
            pltpu.SemaphoreType.DMA,
            pltpu.SemaphoreType.DMA,
            pltpu.SemaphoreType.DMA,
        ],
    )


def _edge_sc(k, qv, src, dst, zrows):
    src4 = src.reshape(_NW, _NG, _GC, _EC)
    dst4 = dst.reshape(_NW, _NG, _GC, _EC)
    return _edge_sc_kernel()(k, qv, src4, dst4, zrows)


def _proj_body(fused, h_ref, wk_ref, wqv_ref, ws_ref, bk_ref, bqv_ref, bs_ref,
               *rest):
    if fused:
        stats_ref, g_ref, bb_ref = rest[0], rest[1], rest[2]
        k_ref, qv_ref, s_ref = rest[3], rest[4], rest[5]
    else:
        k_ref, qv_ref, s_ref = rest[0], rest[1], rest[2]
    h = h_ref[...]
    if fused:
        mean = stats_ref[0:1, :] * (1.0 / _N)
        var = stats_ref[1:2, :] * (1.0 / _N) - mean * mean
        sc = g_ref[...] * lax.rsqrt(var + 1e-5)
        h = (h - mean) * sc + bb_ref[...]
    k_ref[...] = jnp.dot(h, wk_ref[...], preferred_element_type=jnp.float32) + bk_ref[...]
    qv_ref[...] = jnp.dot(h, wqv_ref[...], preferred_element_type=jnp.float32) + bqv_ref[...]
    s_ref[...] = jnp.dot(h, ws_ref[...], preferred_element_type=jnp.float32) + bs_ref[...]


def _proj(h, wk, wqv, ws, bk, bqv, bs, stats=None, g=None, b=None):
    fused = stats is not None
    const = lambda i: (0, 0)
    in_specs = [
        pl.BlockSpec((_RB, _D), lambda i: (i, 0)),
        pl.BlockSpec((_D, _D), const),
        pl.BlockSpec((_D, 2 * _D), const),
        pl.BlockSpec((_D, _D), const),
        pl.BlockSpec((1, _D), const),
        pl.BlockSpec((1, 2 * _D), const),
        pl.BlockSpec((1, _D), const),
    ]
    ins = [h, wk, wqv, ws, bk, bqv, bs]
    if fused:
        in_specs += [pl.BlockSpec((2, _D), const),
                     pl.BlockSpec((1, _D), const),
                     pl.BlockSpec((1, _D), const)]
        ins += [stats, g, b]
    return pl.pallas_call(
        functools.partial(_proj_body, fused),
        grid=(_NB,),
        in_specs=in_specs,
        out_specs=[
            pl.BlockSpec((_RB, _D), lambda i: (i, 0)),
            pl.BlockSpec((_RB, 2 * _D), lambda i: (i, 0)),
            pl.BlockSpec((_RB, _D), lambda i: (i, 0)),
        ],
        out_shape=[
            jax.ShapeDtypeStruct((_N, _D), jnp.float32),
            jax.ShapeDtypeStruct((_N, 2 * _D), jnp.float32),
            jax.ShapeDtypeStruct((_N, _D), jnp.float32),
        ],
    )(*ins)


def _post_body(agg_ref, s_ref, t_ref, stats_ref):
    i = pl.program_id(0)
    t = jnp.maximum(agg_ref[0] + agg_ref[1] + s_ref[...], 0.0)
    t_ref[...] = t

    @pl.when(i == 0)
    def _():
        stats_ref[...] = jnp.zeros_like(stats_ref)

    stats_ref[...] += jnp.concatenate(
        [jnp.sum(t, axis=0, keepdims=True),
         jnp.sum(t * t, axis=0, keepdims=True)], axis=0)


def _post(agg, s):
    return pl.pallas_call(
        _post_body,
        grid=(_NB,),
        in_specs=[
            pl.BlockSpec((_NC, _RB, _D), lambda i: (0, i, 0)),
            pl.BlockSpec((_RB, _D), lambda i: (i, 0)),
        ],
        out_specs=[
            pl.BlockSpec((_RB, _D), lambda i: (i, 0)),
            pl.BlockSpec((2, _D), lambda i: (0, 0)),
        ],
        out_shape=[
            jax.ShapeDtypeStruct((_N, _D), jnp.float32),
            jax.ShapeDtypeStruct((2, _D), jnp.float32),
        ],
    )(agg, s)


def _bn_rows(x, g, b):
    m = jnp.mean(x, axis=0, keepdims=True)
    v = jnp.mean((x - m) * (x - m), axis=0, keepdims=True)
    return (x - m) * lax.rsqrt(v + 1e-5) * g + b


def _final_body(t_ref, stats_ref, bn_g_ref, bn_b_ref, batch_ref,
                gapg_ref, gapb_ref, gspg_ref, gspb_ref,
                w1_ref, b1_ref, lng_ref, lnb_ref, w2_ref, b2_ref,
                out_ref, gap_s, cnt_s, gsp_s):
    i = pl.program_id(0)
    mean = stats_ref[0:1, :] * (1.0 / _N)
    var = stats_ref[1:2, :] * (1.0 / _N) - mean * mean
    sc = bn_g_ref[...] * lax.rsqrt(var + 1e-5)
    h = (t_ref[...] - mean) * sc + bn_b_ref[...]
    bid = batch_ref[...]
    onehot = (bid == lax.broadcasted_iota(jnp.int32, (_RB, _G), 1)
              ).astype(jnp.float32)

    @pl.when(i == 0)
    def _():
        gap_s[...] = jnp.zeros_like(gap_s)
        cnt_s[...] = jnp.zeros_like(cnt_s)
        gsp_s[...] = jnp.full_like(gsp_s, jnp.finfo(jnp.float32).min)

    gap_s[...] += lax.dot_general(onehot, h, (((0,), (0,)), ((), ())),
                                  preferred_element_type=jnp.float32)
    cnt_s[...] += jnp.sum(onehot, axis=0, keepdims=True)

    def gloop(g, c):
        m = jnp.where(bid == g, h, jnp.finfo(jnp.float32).min)
        mx = jnp.max(m, axis=0, keepdims=True)
        gsp_s[pl.ds(g, 1), :] = jnp.maximum(gsp_s[pl.ds(g, 1), :], mx)
        return c

    lax.fori_loop(0, _G, gloop, 0)

    @pl.when(i == _NB - 1)
    def _():
        cnt = jnp.maximum(cnt_s[...].reshape(_G, 1), 1.0)
        gap = _bn_rows(gap_s[...] / cnt, gapg_ref[...], gapb_ref[...])
        gsp = _bn_rows(gsp_s[...], gspg_ref[...], gspb_ref[...])
        cat = jnp.concatenate([gap, gsp], axis=1)
        o = jnp.maximum(
            jnp.dot(cat, w1_ref[...], preferred_element_type=jnp.float32)
            + b1_ref[...], 0.0)
        o = _bn_rows(o, lng_ref[...], lnb_ref[...])
        out_ref[...] = (jnp.dot(o, w2_ref[...],
                                preferred_element_type=jnp.float32)
                        + b2_ref[...])


def _final(t, stats, bn_g, bn_b, batch2, gapg, gapb, gspg, gspb,
           w1, b1, lng, lnb, w2, b2):
    const = lambda i: (0, 0)
    return pl.pallas_call(
        _final_body,
        grid=(_NB,),
        in_specs=[
            pl.BlockSpec((_RB, _D), lambda i: (i, 0)),
            pl.BlockSpec((2, _D), const),
            pl.BlockSpec((1, _D), const),
            pl.BlockSpec((1, _D), const),
            pl.BlockSpec((_RB, 1), lambda i: (i, 0)),
            pl.BlockSpec((1, _D), const),
            pl.BlockSpec((1, _D), const),
            pl.BlockSpec((1, _D), const),
            pl.BlockSpec((1, _D), const),
            pl.BlockSpec((2 * _D, _D), const),
            pl.BlockSpec((1, _D), const),
            pl.BlockSpec((1, _D), const),
            pl.BlockSpec((1, _D), const),
            pl.BlockSpec((_D, _D), const),
            pl.BlockSpec((1, _D), const),
        ],
        out_specs=pl.BlockSpec((_G, _D), const),
        out_shape=jax.ShapeDtypeStruct((_G, _D), jnp.float32),
        scratch_shapes=[
            pltpu.VMEM((_G, _D), jnp.float32),
            pltpu.VMEM((1, _G), jnp.float32),
            pltpu.VMEM((_G, _D), jnp.float32),
        ],
    )(t, stats, bn_g, bn_b, batch2, gapg, gapb, gspg, gspb,
      w1, b1, lng, lnb, w2, b2)


def _layer_params(p, l):
    wk = p[f'conv{l}_Wk']
    wqv = jnp.concatenate([p[f'conv{l}_Wq'], p[f'conv{l}_Wv']], axis=1)
    ws = p[f'conv{l}_Ws']
    bk = p[f'conv{l}_bk'].reshape(1, _D)
    bqv = jnp.concatenate([p[f'conv{l}_bq'], p[f'conv{l}_bv']]).reshape(1, 2 * _D)
    bs = p[f'conv{l}_b'].reshape(1, _D)
    return wk, wqv, ws, bk, bqv, bs


def kernel(x, edge_index, batch, params):
    p = params
    src = edge_index[0]
    dst = edge_index[1]
    batch2 = batch.reshape(_N, 1)
    zrows = jnp.zeros((_ZR, _D), jnp.float32)

    k0, qv0, s0 = _proj(x, *_layer_params(p, 0))
    agg0 = _edge_sc(k0, qv0, src, dst, zrows)
    t0, st0 = _post(agg0, s0)

    k1, qv1, s1 = _proj(t0, *_layer_params(p, 1), stats=st0,
                        g=p['bn0_g'].reshape(1, _D), b=p['bn0_b'].reshape(1, _D))
    agg1 = _edge_sc(k1, qv1, src, dst, zrows)
    t1, st1 = _post(agg1, s1)

    w2 = jnp.zeros((_D, _D), jnp.float32).at[:, :10].set(p['last_W'])
    b2 = jnp.zeros((1, _D), jnp.float32).at[0, :10].set(p['last_b'])
    out = _final(t1, st1,
                 p['bn1_g'].reshape(1, _D), p['bn1_b'].reshape(1, _D),
                 batch2,
                 p['gap_g'].reshape(1, _D), p['gap_b'].reshape(1, _D),
                 p['gsp_g'].reshape(1, _D), p['gsp_b'].reshape(1, _D),
                 p['hl0_W'], p['hl0_b'].reshape(1, _D),
                 p['hln0_g'].reshape(1, _D), p['hln0_b'].reshape(1, _D),
                 w2, b2)
    return out[:, :10]


# speedup vs baseline: 7.0853x; 3.9862x over previous
import functools

import jax
import jax.numpy as jnp
from jax import lax
from jax.experimental import pallas as pl
from jax.experimental.pallas import tpu as pltpu
from jax.experimental.pallas import tpu_sc as plsc

_N = 10000
_E = 320000
_D = 128
_G = 64

_RB = 1000
_NB = _N // _RB

_NC = 2
_NS = 16
_NW = _NC * _NS
_EPT = _E // _NW
_EC = 40
_NCHUNK = _EPT // _EC
_GC = 50
_NG = _NCHUNK // _GC
_ZR = 40
_NZ = _N // _ZR
_ZT = -(-_NZ // _NS)


def _edge_sc_body(k_hbm, qv_hbm, src_hbm, dst_hbm, z_hbm, out_hbm,
                  sidx, didx, kb0, kb1, kb2, qb0, qb1, acc,
                  sg0, sg1, ss0, ss1, ss2):
    cid = lax.axis_index("c")
    sid = lax.axis_index("s")
    wid = sid * _NC + cid
    kb = (kb0, kb1, kb2)
    qb = (qb0, qb1)
    sg = (sg0, sg1)
    ss = (ss0, ss1, ss2)

    pltpu.sync_copy(z_hbm, kb0)
    for t in range(_ZT):
        ci = sid + _NS * t

        @pl.when(ci < _NZ)
        def _():
            off = pl.multiple_of(ci * _ZR, 8)
            pltpu.sync_copy(kb0, acc.at[pl.ds(off, _ZR)])

    plsc.subcore_barrier()

    def issue_gather(c, m, p):
        pltpu.async_copy(k_hbm.at[didx.at[c]], kb[m], sg[p])
        pltpu.async_copy(qv_hbm.at[sidx.at[c]], qb[p], sg[p])

    def drain_gather(c, m, p):
        pltpu.make_async_copy(k_hbm.at[didx.at[c]], kb[m], sg[p]).wait()
        pltpu.make_async_copy(qv_hbm.at[sidx.at[c]], qb[p], sg[p]).wait()

    def drain_scatter(c, m):
        pltpu.make_async_copy(kb[m], acc.at[didx.at[c]], ss[m]).wait()

    def compute(m, p):
        kbp, qbp = kb[m], qb[p]

        def row(r2, rc):
            for rr in range(2):
                r = 2 * r2 + rr
                vals = []
                for j in range(_D // 16):
                    kk = kbp[r, pl.ds(16 * j, 16)]
                    qq = qbp[r, pl.ds(16 * j, 16)]
                    vv = qbp[r, pl.ds(_D + 16 * j, 16)]
                    vals.append(vv / (1.0 + jnp.exp(-(kk + qq))))
                for j in range(_D // 16):
                    kbp[r, pl.ds(16 * j, 16)] = vals[j]
            return rc

        lax.fori_loop(0, _EC // 2, row, 0)

    def step(c, m, p, issue_next, guard_drain):
        mm = (m + 1) % 3

        def _drain_prev():
            drain_scatter(c - 2, mm)

        if guard_drain:
            pl.when(c >= 2)(_drain_prev)
        else:
            _drain_prev()
        if issue_next:
            issue_gather(c + 1, mm, 1 - p)
        drain_gather(c, m, p)
        compute(m, p)
        pltpu.async_copy(kb[m], acc.at[didx.at[c]], ss[m], add=True)

    def six(t, carry):
        for u in range(6):
            step(6 * t + u, u % 3, u % 2, True, True)
        return carry

    def group(g, carry):
        pltpu.sync_copy(src_hbm.at[wid, g], sidx)
        pltpu.sync_copy(dst_hbm.at[wid, g], didx)
        issue_gather(0, 0, 0)
        lax.fori_loop(0, (_GC - 2) // 6, six, 0)
        step(_GC - 2, (_GC - 2) % 3, (_GC - 2) % 2, True, False)
        step(_GC - 1, (_GC - 1) % 3, (_GC - 1) % 2, False, False)
        drain_scatter(_GC - 2, (_GC - 2) % 3)
        drain_scatter(_GC - 1, (_GC - 1) % 3)
        return carry

    lax.fori_loop(0, _NG, group, 0)

    plsc.subcore_barrier()

    for t in range(_ZT):
        ci = sid + _NS * t

        @pl.when(ci < _NZ)
        def _():
            off = pl.multiple_of(ci * _ZR, 8)
            pltpu.sync_copy(acc.at[pl.ds(off, _ZR)], kb0)
            pltpu.sync_copy(kb0, out_hbm.at[cid, pl.ds(off, _ZR)])


@functools.cache
def _edge_sc_kernel():
    return pl.kernel(
        _edge_sc_body,
        out_type=jax.ShapeDtypeStruct((_NC, _N, _D), jnp.float32),
        mesh=plsc.VectorSubcoreMesh(core_axis_name="c", subcore_axis_name="s",
                                    num_cores=_NC, num_subcores=_NS),
        scratch_types=[
            pltpu.VMEM((_GC, _EC), jnp.int32),
            pltpu.VMEM((_GC, _EC), jnp.int32),
            pltpu.VMEM((_EC, _D), jnp.float32),
            pltpu.VMEM((_EC, _D), jnp.float32),
            pltpu.VMEM((_EC, _D), jnp.float32),
            pltpu.VMEM((_EC, 2 * _D), jnp.float32),
            pltpu.VMEM((_EC, 2 * _D), jnp.float32),
            pltpu.VMEM_SHARED((_N, _D), jnp.float32),
            pltpu.SemaphoreType.DMA,
            pltpu.SemaphoreType.DMA,
            pltpu.SemaphoreType.DMA,
            pltpu.SemaphoreType.DMA,
            pltpu.SemaphoreType.DMA,
        ],
    )


def _edge_sc(k, qv, src, dst, zrows):
    src4 = src.reshape(_NW, _NG, _GC, _EC)
    dst4 = dst.reshape(_NW, _NG, _GC, _EC)
    return _edge_sc_kernel()(k, qv, src4, dst4, zrows)


def _proj_body(fused, h_ref, wk_ref, wqv_ref, ws_ref, bk_ref, bqv_ref, bs_ref,
               *rest):
    if fused:
        stats_ref, g_ref, bb_ref = rest[0], rest[1], rest[2]
        k_ref, qv_ref, s_ref = rest[3], rest[4], rest[5]
    else:
        k_ref, qv_ref, s_ref = rest[0], rest[1], rest[2]
    h = h_ref[...]
    if fused:
        mean = stats_ref[0:1, :] * (1.0 / _N)
        var = stats_ref[1:2, :] * (1.0 / _N) - mean * mean
        sc = g_ref[...] * lax.rsqrt(var + 1e-5)
        h = (h - mean) * sc + bb_ref[...]
    k_ref[...] = jnp.dot(h, wk_ref[...], preferred_element_type=jnp.float32) + bk_ref[...]
    qv_ref[...] = jnp.dot(h, wqv_ref[...], preferred_element_type=jnp.float32) + bqv_ref[...]
    s_ref[...] = jnp.dot(h, ws_ref[...], preferred_element_type=jnp.float32) + bs_ref[...]


def _proj(h, wk, wqv, ws, bk, bqv, bs, stats=None, g=None, b=None):
    fused = stats is not None
    const = lambda i: (0, 0)
    in_specs = [
        pl.BlockSpec((_RB, _D), lambda i: (i, 0)),
        pl.BlockSpec((_D, _D), const),
        pl.BlockSpec((_D, 2 * _D), const),
        pl.BlockSpec((_D, _D), const),
        pl.BlockSpec((1, _D), const),
        pl.BlockSpec((1, 2 * _D), const),
        pl.BlockSpec((1, _D), const),
    ]
    ins = [h, wk, wqv, ws, bk, bqv, bs]
    if fused:
        in_specs += [pl.BlockSpec((2, _D), const),
                     pl.BlockSpec((1, _D), const),
                     pl.BlockSpec((1, _D), const)]
        ins += [stats, g, b]
    return pl.pallas_call(
        functools.partial(_proj_body, fused),
        grid=(_NB,),
        in_specs=in_specs,
        out_specs=[
            pl.BlockSpec((_RB, _D), lambda i: (i, 0)),
            pl.BlockSpec((_RB, 2 * _D), lambda i: (i, 0)),
            pl.BlockSpec((_RB, _D), lambda i: (i, 0)),
        ],
        out_shape=[
            jax.ShapeDtypeStruct((_N, _D), jnp.float32),
            jax.ShapeDtypeStruct((_N, 2 * _D), jnp.float32),
            jax.ShapeDtypeStruct((_N, _D), jnp.float32),
        ],
    )(*ins)


def _post_body(agg_ref, s_ref, t_ref, stats_ref):
    i = pl.program_id(0)
    t = jnp.maximum(agg_ref[0] + agg_ref[1] + s_ref[...], 0.0)
    t_ref[...] = t

    @pl.when(i == 0)
    def _():
        stats_ref[...] = jnp.zeros_like(stats_ref)

    stats_ref[...] += jnp.concatenate(
        [jnp.sum(t, axis=0, keepdims=True),
         jnp.sum(t * t, axis=0, keepdims=True)], axis=0)


def _post(agg, s):
    return pl.pallas_call(
        _post_body,
        grid=(_NB,),
        in_specs=[
            pl.BlockSpec((_NC, _RB, _D), lambda i: (0, i, 0)),
            pl.BlockSpec((_RB, _D), lambda i: (i, 0)),
        ],
        out_specs=[
            pl.BlockSpec((_RB, _D), lambda i: (i, 0)),
            pl.BlockSpec((2, _D), lambda i: (0, 0)),
        ],
        out_shape=[
            jax.ShapeDtypeStruct((_N, _D), jnp.float32),
            jax.ShapeDtypeStruct((2, _D), jnp.float32),
        ],
    )(agg, s)


def _bn_rows(x, g, b):
    m = jnp.mean(x, axis=0, keepdims=True)
    v = jnp.mean((x - m) * (x - m), axis=0, keepdims=True)
    return (x - m) * lax.rsqrt(v + 1e-5) * g + b


def _final_body(t_ref, stats_ref, bn_g_ref, bn_b_ref, batch_ref,
                gapg_ref, gapb_ref, gspg_ref, gspb_ref,
                w1_ref, b1_ref, lng_ref, lnb_ref, w2_ref, b2_ref,
                out_ref, gap_s, cnt_s, gsp_s):
    i = pl.program_id(0)
    mean = stats_ref[0:1, :] * (1.0 / _N)
    var = stats_ref[1:2, :] * (1.0 / _N) - mean * mean
    sc = bn_g_ref[...] * lax.rsqrt(var + 1e-5)
    h = (t_ref[...] - mean) * sc + bn_b_ref[...]
    bid = batch_ref[...]
    onehot = (bid == lax.broadcasted_iota(jnp.int32, (_RB, _G), 1)
              ).astype(jnp.float32)

    @pl.when(i == 0)
    def _():
        gap_s[...] = jnp.zeros_like(gap_s)
        cnt_s[...] = jnp.zeros_like(cnt_s)
        gsp_s[...] = jnp.full_like(gsp_s, jnp.finfo(jnp.float32).min)

    gap_s[...] += lax.dot_general(onehot, h, (((0,), (0,)), ((), ())),
                                  preferred_element_type=jnp.float32)
    cnt_s[...] += jnp.sum(onehot, axis=0, keepdims=True)

    def gloop(g, c):
        m = jnp.where(bid == g, h, jnp.finfo(jnp.float32).min)
        mx = jnp.max(m, axis=0, keepdims=True)
        gsp_s[pl.ds(g, 1), :] = jnp.maximum(gsp_s[pl.ds(g, 1), :], mx)
        return c

    lax.fori_loop(0, _G, gloop, 0)

    @pl.when(i == _NB - 1)
    def _():
        cnt = jnp.maximum(cnt_s[...].reshape(_G, 1), 1.0)
        gap = _bn_rows(gap_s[...] / cnt, gapg_ref[...], gapb_ref[...])
        gsp = _bn_rows(gsp_s[...], gspg_ref[...], gspb_ref[...])
        cat = jnp.concatenate([gap, gsp], axis=1)
        o = jnp.maximum(
            jnp.dot(cat, w1_ref[...], preferred_element_type=jnp.float32)
            + b1_ref[...], 0.0)
        o = _bn_rows(o, lng_ref[...], lnb_ref[...])
        out_ref[...] = (jnp.dot(o, w2_ref[...],
                                preferred_element_type=jnp.float32)
                        + b2_ref[...])


def _final(t, stats, bn_g, bn_b, batch2, gapg, gapb, gspg, gspb,
           w1, b1, lng, lnb, w2, b2):
    const = lambda i: (0, 0)
    return pl.pallas_call(
        _final_body,
        grid=(_NB,),
        in_specs=[
            pl.BlockSpec((_RB, _D), lambda i: (i, 0)),
            pl.BlockSpec((2, _D), const),
            pl.BlockSpec((1, _D), const),
            pl.BlockSpec((1, _D), const),
            pl.BlockSpec((_RB, 1), lambda i: (i, 0)),
            pl.BlockSpec((1, _D), const),
            pl.BlockSpec((1, _D), const),
            pl.BlockSpec((1, _D), const),
            pl.BlockSpec((1, _D), const),
            pl.BlockSpec((2 * _D, _D), const),
            pl.BlockSpec((1, _D), const),
            pl.BlockSpec((1, _D), const),
            pl.BlockSpec((1, _D), const),
            pl.BlockSpec((_D, _D), const),
            pl.BlockSpec((1, _D), const),
        ],
        out_specs=pl.BlockSpec((_G, _D), const),
        out_shape=jax.ShapeDtypeStruct((_G, _D), jnp.float32),
        scratch_shapes=[
            pltpu.VMEM((_G, _D), jnp.float32),
            pltpu.VMEM((1, _G), jnp.float32),
            pltpu.VMEM((_G, _D), jnp.float32),
        ],
    )(t, stats, bn_g, bn_b, batch2, gapg, gapb, gspg, gspb,
      w1, b1, lng, lnb, w2, b2)


def _layer_params(p, l):
    wk = p[f'conv{l}_Wk']
    wqv = jnp.concatenate([p[f'conv{l}_Wq'], p[f'conv{l}_Wv']], axis=1)
    ws = p[f'conv{l}_Ws']
    bk = p[f'conv{l}_bk'].reshape(1, _D)
    bqv = jnp.concatenate([p[f'conv{l}_bq'], p[f'conv{l}_bv']]).reshape(1, 2 * _D)
    bs = p[f'conv{l}_b'].reshape(1, _D)
    return wk, wqv, ws, bk, bqv, bs


def kernel(x, edge_index, batch, params):
    p = params
    src = edge_index[0]
    dst = edge_index[1]
    batch2 = batch.reshape(_N, 1)
    zrows = jnp.zeros((_ZR, _D), jnp.float32)

    k0, qv0, s0 = _proj(x, *_layer_params(p, 0))
    agg0 = _edge_sc(k0, qv0, src, dst, zrows)
    t0, st0 = _post(agg0, s0)

    k1, qv1, s1 = _proj(t0, *_layer_params(p, 1), stats=st0,
                        g=p['bn0_g'].reshape(1, _D), b=p['bn0_b'].reshape(1, _D))
    agg1 = _edge_sc(k1, qv1, src, dst, zrows)
    t1, st1 = _post(agg1, s1)

    w2 = jnp.zeros((_D, _D), jnp.float32).at[:, :10].set(p['last_W'])
    b2 = jnp.zeros((1, _D), jnp.float32).at[0, :10].set(p['last_b'])
    out = _final(t1, st1,
                 p['bn1_g'].reshape(1, _D), p['bn1_b'].reshape(1, _D),
                 batch2,
                 p['gap_g'].reshape(1, _D), p['gap_b'].reshape(1, _D),
                 p['gsp_g'].reshape(1, _D), p['gsp_b'].reshape(1, _D),
                 p['hl0_W'], p['hl0_b'].reshape(1, _D),
                 p['hln0_g'].reshape(1, _D), p['hln0_b'].reshape(1, _D),
                 w2, b2)
    return out[:, :10]


# speedup vs baseline: 8.2352x; 1.1623x over previous
import functools

import jax
import jax.numpy as jnp
from jax import lax
from jax.experimental import pallas as pl
from jax.experimental.pallas import tpu as pltpu
from jax.experimental.pallas import tpu_sc as plsc

_N = 10000
_E = 320000
_D = 128
_G = 64

_RB = 1000
_NB = _N // _RB

_NC = 2
_NS = 16
_NW = _NC * _NS
_EPT = _E // _NW
_EC = 40
_NCHUNK = _EPT // _EC
_GC = 50
_NG = _NCHUNK // _GC
_ZR = 40
_NZ = _N // _ZR
_ZT = -(-_NZ // _NS)


def _edge_sc_body(k_hbm, qv_hbm, src_hbm, dst_hbm, z_hbm, out_hbm,
                  sidx, didx, kb0, kb1, kb2, qb0, qb1, acc,
                  sg0, sg1, ss0, ss1, ss2):
    cid = lax.axis_index("c")
    sid = lax.axis_index("s")
    wid = sid * _NC + cid
    kb = (kb0, kb1, kb2)
    qb = (qb0, qb1)
    sg = (sg0, sg1)
    ss = (ss0, ss1, ss2)

    pltpu.sync_copy(z_hbm, kb0)
    for t in range(_ZT):
        ci = sid + _NS * t

        @pl.when(ci < _NZ)
        def _():
            off = pl.multiple_of(ci * _ZR, 8)
            pltpu.sync_copy(kb0, acc.at[pl.ds(off, _ZR)])

    plsc.subcore_barrier()

    def issue_gather(c, m, p):
        pltpu.async_copy(k_hbm.at[didx.at[c]], kb[m], sg[p])
        pltpu.async_copy(qv_hbm.at[sidx.at[c]], qb[p], sg[p])

    def drain_gather(c, m, p):
        pltpu.make_async_copy(k_hbm.at[didx.at[c]], kb[m], sg[p]).wait()
        pltpu.make_async_copy(qv_hbm.at[sidx.at[c]], qb[p], sg[p]).wait()

    def drain_scatter(c, m):
        pltpu.make_async_copy(kb[m], acc.at[didx.at[c]], ss[m]).wait()

    def compute(m, p):
        kbp, qbp = kb[m], qb[p]

        def row(r2, rc):
            for rr in range(2):
                r = 2 * r2 + rr
                vals = []
                for j in range(_D // 16):
                    kk = kbp[r, pl.ds(16 * j, 16)]
                    qq = qbp[r, pl.ds(16 * j, 16)]
                    vv = qbp[r, pl.ds(_D + 16 * j, 16)]
                    vals.append(vv / (1.0 + jnp.exp(-(kk + qq))))
                for j in range(_D // 16):
                    kbp[r, pl.ds(16 * j, 16)] = vals[j]
            return rc

        lax.fori_loop(0, _EC // 2, row, 0)

    def step(c, m, p, issue_next, guard_drain):
        mm = (m + 1) % 3

        def _drain_prev():
            drain_scatter(c - 2, mm)

        if guard_drain:
            pl.when(c >= 2)(_drain_prev)
        else:
            _drain_prev()
        if issue_next:
            issue_gather(c + 1, mm, 1 - p)
        drain_gather(c, m, p)
        compute(m, p)
        pltpu.async_copy(kb[m], acc.at[didx.at[c]], ss[m], add=True)

    def six(t, carry):
        for u in range(6):
            step(6 * t + u, u % 3, u % 2, True, True)
        return carry

    def group(g, carry):
        pltpu.sync_copy(src_hbm.at[wid, g], sidx)
        pltpu.sync_copy(dst_hbm.at[wid, g], didx)
        issue_gather(0, 0, 0)
        lax.fori_loop(0, (_GC - 2) // 6, six, 0)
        step(_GC - 2, (_GC - 2) % 3, (_GC - 2) % 2, True, False)
        step(_GC - 1, (_GC - 1) % 3, (_GC - 1) % 2, False, False)
        drain_scatter(_GC - 2, (_GC - 2) % 3)
        drain_scatter(_GC - 1, (_GC - 1) % 3)
        return carry

    lax.fori_loop(0, _NG, group, 0)

    plsc.subcore_barrier()

    for t in range(_ZT):
        ci = sid + _NS * t

        @pl.when(ci < _NZ)
        def _():
            off = pl.multiple_of(ci * _ZR, 8)
            pltpu.sync_copy(acc.at[pl.ds(off, _ZR)], kb0)
            pltpu.sync_copy(kb0, out_hbm.at[cid, pl.ds(off, _ZR)])


@functools.cache
def _edge_sc_kernel():
    return pl.kernel(
        _edge_sc_body,
        out_type=jax.ShapeDtypeStruct((_NC, _N, _D), jnp.float32),
        mesh=plsc.VectorSubcoreMesh(core_axis_name="c", subcore_axis_name="s",
                                    num_cores=_NC, num_subcores=_NS),
        scratch_types=[
            pltpu.VMEM((_GC, _EC), jnp.int32),
            pltpu.VMEM((_GC, _EC), jnp.int32),
            pltpu.VMEM((_EC, _D), jnp.float32),
            pltpu.VMEM((_EC, _D), jnp.float32),
            pltpu.VMEM((_EC, _D), jnp.float32),
            pltpu.VMEM((_EC, 2 * _D), jnp.float32),
            pltpu.VMEM((_EC, 2 * _D), jnp.float32),
            pltpu.VMEM_SHARED((_N, _D), jnp.float32),
            pltpu.SemaphoreType.DMA,
            pltpu.SemaphoreType.DMA,
            pltpu.SemaphoreType.DMA,
            pltpu.SemaphoreType.DMA,
            pltpu.SemaphoreType.DMA,
        ],
    )


def _edge_sc(k, qv, src, dst, zrows):
    src4 = src.reshape(_NW, _NG, _GC, _EC)
    dst4 = dst.reshape(_NW, _NG, _GC, _EC)
    return _edge_sc_kernel()(k, qv, src4, dst4, zrows)


def _proj_body(fused, h_ref, wk_ref, wqv_ref, ws_ref, bk_ref, bqv_ref, bs_ref,
               *rest):
    if fused:
        stats_ref, g_ref, bb_ref = rest[0], rest[1], rest[2]
        k_ref, qv_ref, s_ref = rest[3], rest[4], rest[5]
    else:
        k_ref, qv_ref, s_ref = rest[0], rest[1], rest[2]
    h = h_ref[...]
    if fused:
        mean = stats_ref[0:1, :] * (1.0 / _N)
        var = stats_ref[1:2, :] * (1.0 / _N) - mean * mean
        sc = g_ref[...] * lax.rsqrt(var + 1e-5)
        h = (h - mean) * sc + bb_ref[...]
    k_ref[...] = jnp.dot(h, wk_ref[...], preferred_element_type=jnp.float32) + bk_ref[...]
    qv_ref[...] = jnp.dot(h, wqv_ref[...], preferred_element_type=jnp.float32) + bqv_ref[...]
    s_ref[...] = jnp.dot(h, ws_ref[...], preferred_element_type=jnp.float32) + bs_ref[...]


def _proj(h, wk, wqv, ws, bk, bqv, bs, stats=None, g=None, b=None):
    fused = stats is not None
    const = lambda i: (0, 0)
    in_specs = [
        pl.BlockSpec((_RB, _D), lambda i: (i, 0)),
        pl.BlockSpec((_D, _D), const),
        pl.BlockSpec((_D, 2 * _D), const),
        pl.BlockSpec((_D, _D), const),
        pl.BlockSpec((1, _D), const),
        pl.BlockSpec((1, 2 * _D), const),
        pl.BlockSpec((1, _D), const),
    ]
    ins = [h, wk, wqv, ws, bk, bqv, bs]
    if fused:
        in_specs += [pl.BlockSpec((2, _D), const),
                     pl.BlockSpec((1, _D), const),
                     pl.BlockSpec((1, _D), const)]
        ins += [stats, g, b]
    return pl.pallas_call(
        functools.partial(_proj_body, fused),
        grid=(_NB,),
        in_specs=in_specs,
        out_specs=[
            pl.BlockSpec((_RB, _D), lambda i: (i, 0)),
            pl.BlockSpec((_RB, 2 * _D), lambda i: (i, 0)),
            pl.BlockSpec((_RB, _D), lambda i: (i, 0)),
        ],
        out_shape=[
            jax.ShapeDtypeStruct((_N, _D), jnp.float32),
            jax.ShapeDtypeStruct((_N, 2 * _D), jnp.float32),
            jax.ShapeDtypeStruct((_N, _D), jnp.float32),
        ],
    )(*ins)


def _postproj_body(agg_ref, s_ref, wk_ref, wqv_ref, ws_ref,
                   bk_ref, bqv_ref, bs_ref, g_ref, bb_ref,
                   k_ref, qv_ref, s_out_ref, tbuf, stats):
    i = pl.program_id(0)
    t = jnp.maximum(agg_ref[0] + agg_ref[1] + s_ref[...], 0.0)
    tbuf[pl.ds(i * _RB, _RB), :] = t

    @pl.when(i == 0)
    def _():
        stats[...] = jnp.zeros_like(stats)

    stats[...] += jnp.concatenate(
        [jnp.sum(t, axis=0, keepdims=True),
         jnp.sum(t * t, axis=0, keepdims=True)], axis=0)

    @pl.when(i == _NB - 1)
    def _():
        mean = stats[0:1, :] * (1.0 / _N)
        var = stats[1:2, :] * (1.0 / _N) - mean * mean
        sc = g_ref[...] * lax.rsqrt(var + 1e-5)
        sh = bb_ref[...] - mean * sc
        for b in range(_NB):
            h = tbuf[pl.ds(b * _RB, _RB), :] * sc + sh
            k_ref[pl.ds(b * _RB, _RB), :] = jnp.dot(
                h, wk_ref[...], preferred_element_type=jnp.float32) + bk_ref[...]
            qv_ref[pl.ds(b * _RB, _RB), :] = jnp.dot(
                h, wqv_ref[...], preferred_element_type=jnp.float32) + bqv_ref[...]
            s_out_ref[pl.ds(b * _RB, _RB), :] = jnp.dot(
                h, ws_ref[...], preferred_element_type=jnp.float32) + bs_ref[...]


def _postproj(agg, s, wk, wqv, ws, bk, bqv, bs, g, b):
    const = lambda i: (0, 0)
    return pl.pallas_call(
        _postproj_body,
        grid=(_NB,),
        in_specs=[
            pl.BlockSpec((_NC, _RB, _D), lambda i: (0, i, 0)),
            pl.BlockSpec((_RB, _D), lambda i: (i, 0)),
            pl.BlockSpec((_D, _D), const),
            pl.BlockSpec((_D, 2 * _D), const),
            pl.BlockSpec((_D, _D), const),
            pl.BlockSpec((1, _D), const),
            pl.BlockSpec((1, 2 * _D), const),
            pl.BlockSpec((1, _D), const),
            pl.BlockSpec((1, _D), const),
            pl.BlockSpec((1, _D), const),
        ],
        out_specs=[
            pl.BlockSpec((_N, _D), const),
            pl.BlockSpec((_N, 2 * _D), const),
            pl.BlockSpec((_N, _D), const),
        ],
        out_shape=[
            jax.ShapeDtypeStruct((_N, _D), jnp.float32),
            jax.ShapeDtypeStruct((_N, 2 * _D), jnp.float32),
            jax.ShapeDtypeStruct((_N, _D), jnp.float32),
        ],
        scratch_shapes=[
            pltpu.VMEM((_N, _D), jnp.float32),
            pltpu.VMEM((2, _D), jnp.float32),
        ],
    )(agg, s, wk, wqv, ws, bk, bqv, bs, g, b)


def _bn_rows(x, g, b):
    m = jnp.mean(x, axis=0, keepdims=True)
    v = jnp.mean((x - m) * (x - m), axis=0, keepdims=True)
    return (x - m) * lax.rsqrt(v + 1e-5) * g + b


def _final_body(agg_ref, s_ref, bn_g_ref, bn_b_ref, batch_ref,
                gapg_ref, gapb_ref, gspg_ref, gspb_ref,
                w1_ref, b1_ref, lng_ref, lnb_ref, w2_ref, b2_ref,
                out_ref, tbuf, stats, gap_s, cnt_s, gsp_s):
    i = pl.program_id(0)
    t = jnp.maximum(agg_ref[0] + agg_ref[1] + s_ref[...], 0.0)
    tbuf[pl.ds(i * _RB, _RB), :] = t

    @pl.when(i == 0)
    def _():
        stats[...] = jnp.zeros_like(stats)

    stats[...] += jnp.concatenate(
        [jnp.sum(t, axis=0, keepdims=True),
         jnp.sum(t * t, axis=0, keepdims=True)], axis=0)

    @pl.when(i == _NB - 1)
    def _():
        mean = stats[0:1, :] * (1.0 / _N)
        var = stats[1:2, :] * (1.0 / _N) - mean * mean
        sc = bn_g_ref[...] * lax.rsqrt(var + 1e-5)
        sh = bn_b_ref[...] - mean * sc
        gap_s[...] = jnp.zeros_like(gap_s)
        cnt_s[...] = jnp.zeros_like(cnt_s)
        gsp_s[...] = jnp.full_like(gsp_s, jnp.finfo(jnp.float32).min)
        for b in range(_NB):
            h = tbuf[pl.ds(b * _RB, _RB), :] * sc + sh
            bid = batch_ref[pl.ds(b * _RB, _RB), :]
            onehot = (bid == lax.broadcasted_iota(jnp.int32, (_RB, _G), 1)
                      ).astype(jnp.float32)
            gap_s[...] += lax.dot_general(onehot, h, (((0,), (0,)), ((), ())),
                                          preferred_element_type=jnp.float32)
            cnt_s[...] += jnp.sum(onehot, axis=0, keepdims=True)

            def gloop(g, c):
                m = jnp.where(bid == g, h, jnp.finfo(jnp.float32).min)
                mx = jnp.max(m, axis=0, keepdims=True)
                gsp_s[pl.ds(g, 1), :] = jnp.maximum(gsp_s[pl.ds(g, 1), :], mx)
                return c

            lax.fori_loop(bid[0, 0], bid[_RB - 1, 0] + 1, gloop, 0)

        cnt = jnp.maximum(cnt_s[...].reshape(_G, 1), 1.0)
        gap = _bn_rows(gap_s[...] / cnt, gapg_ref[...], gapb_ref[...])
        gsp = _bn_rows(gsp_s[...], gspg_ref[...], gspb_ref[...])
        cat = jnp.concatenate([gap, gsp], axis=1)
        o = jnp.maximum(
            jnp.dot(cat, w1_ref[...], preferred_element_type=jnp.float32)
            + b1_ref[...], 0.0)
        o = _bn_rows(o, lng_ref[...], lnb_ref[...])
        out_ref[...] = (jnp.dot(o, w2_ref[...],
                                preferred_element_type=jnp.float32)
                        + b2_ref[...])


def _final(agg, s, bn_g, bn_b, batch2, gapg, gapb, gspg, gspb,
           w1, b1, lng, lnb, w2, b2):
    const = lambda i: (0, 0)
    return pl.pallas_call(
        _final_body,
        grid=(_NB,),
        in_specs=[
            pl.BlockSpec((_NC, _RB, _D), lambda i: (0, i, 0)),
            pl.BlockSpec((_RB, _D), lambda i: (i, 0)),
            pl.BlockSpec((1, _D), const),
            pl.BlockSpec((1, _D), const),
            pl.BlockSpec((_N, 1), const),
            pl.BlockSpec((1, _D), const),
            pl.BlockSpec((1, _D), const),
            pl.BlockSpec((1, _D), const),
            pl.BlockSpec((1, _D), const),
            pl.BlockSpec((2 * _D, _D), const),
            pl.BlockSpec((1, _D), const),
            pl.BlockSpec((1, _D), const),
            pl.BlockSpec((1, _D), const),
            pl.BlockSpec((_D, _D), const),
            pl.BlockSpec((1, _D), const),
        ],
        out_specs=pl.BlockSpec((_G, _D), const),
        out_shape=jax.ShapeDtypeStruct((_G, _D), jnp.float32),
        scratch_shapes=[
            pltpu.VMEM((_N, _D), jnp.float32),
            pltpu.VMEM((2, _D), jnp.float32),
            pltpu.VMEM((_G, _D), jnp.float32),
            pltpu.VMEM((1, _G), jnp.float32),
            pltpu.VMEM((_G, _D), jnp.float32),
        ],
    )(agg, s, bn_g, bn_b, batch2, gapg, gapb, gspg, gspb,
      w1, b1, lng, lnb, w2, b2)


def _layer_params(p, l):
    wk = p[f'conv{l}_Wk']
    wqv = jnp.concatenate([p[f'conv{l}_Wq'], p[f'conv{l}_Wv']], axis=1)
    ws = p[f'conv{l}_Ws']
    bk = p[f'conv{l}_bk'].reshape(1, _D)
    bqv = jnp.concatenate([p[f'conv{l}_bq'], p[f'conv{l}_bv']]).reshape(1, 2 * _D)
    bs = p[f'conv{l}_b'].reshape(1, _D)
    return wk, wqv, ws, bk, bqv, bs


def kernel(x, edge_index, batch, params):
    p = params
    src = edge_index[0]
    dst = edge_index[1]
    batch2 = batch.reshape(_N, 1)
    zrows = jnp.zeros((_ZR, _D), jnp.float32)

    k0, qv0, s0 = _proj(x, *_layer_params(p, 0))
    agg0 = _edge_sc(k0, qv0, src, dst, zrows)

    k1, qv1, s1 = _postproj(agg0, s0, *_layer_params(p, 1),
                            p['bn0_g'].reshape(1, _D), p['bn0_b'].reshape(1, _D))
    agg1 = _edge_sc(k1, qv1, src, dst, zrows)

    w2 = jnp.zeros((_D, _D), jnp.float32).at[:, :10].set(p['last_W'])
    b2 = jnp.zeros((1, _D), jnp.float32).at[0, :10].set(p['last_b'])
    out = _final(agg1, s1,
                 p['bn1_g'].reshape(1, _D), p['bn1_b'].reshape(1, _D),
                 batch2,
                 p['gap_g'].reshape(1, _D), p['gap_b'].reshape(1, _D),
                 p['gsp_g'].reshape(1, _D), p['gsp_b'].reshape(1, _D),
                 p['hl0_W'], p['hl0_b'].reshape(1, _D),
                 p['hln0_g'].reshape(1, _D), p['hln0_b'].reshape(1, _D),
                 w2, b2)
    return out[:, :10]
